# Initial kernel scaffold; baseline (speedup 1.0000x reference)
import jax
import jax.numpy as jnp
from jax.experimental import pallas as pl


def kernel(x, packed_w, packed_b):
    raise NotImplementedError("write your pallas kernel here")


# speedup vs baseline: 6.8772x; 6.8772x over previous
import functools

import jax
import jax.numpy as jnp
from jax import lax
from jax.experimental import pallas as pl
from jax.experimental.pallas import tpu as pltpu


def _drnn_body(x_ref, wd_ref, we_ref, b_ref, o_ref, p_ref, f_ref, e_ref,
               *, H, S, CP, n_sub):
    C = CP // 8
    M = H * S

    p_ref[...] = jnp.zeros_like(p_ref)
    f_ref[...] = jnp.zeros_like(f_ref)

    def conv3x3(h_val, li, relu):
        p_ref[1:H + 1] = h_val.astype(jnp.bfloat16)
        f_ref[1:H + 1, 1:S, 0:C] = p_ref[1:H + 1, 0:S - 1, CP - C:CP]
        f_ref[1:H + 1, 0:S - 1, C:2 * C] = p_ref[1:H + 1, 1:S, 0:C]
        for dy in range(3):
            e_ref[:, :, dy * 2 * C:(dy + 1) * 2 * C] = f_ref[dy:dy + H]
        acc = jnp.dot(p_ref[0:H].reshape(M, CP), wd_ref[li, 0],
                      preferred_element_type=jnp.float32)
        acc = acc + jnp.dot(p_ref[1:H + 1].reshape(M, CP), wd_ref[li, 1],
                            preferred_element_type=jnp.float32)
        acc = acc + jnp.dot(p_ref[2:H + 2].reshape(M, CP), wd_ref[li, 2],
                            preferred_element_type=jnp.float32)
        acc = acc + jnp.dot(e_ref[...].reshape(M, 6 * C), we_ref[li],
                            preferred_element_type=jnp.float32)
        acc = acc + b_ref[li]
        if relu:
            acc = jnp.maximum(acc, 0.0)
        return acc.reshape(H, S, CP)

    def subnet(si, h_in):
        base = si * 7
        out1 = conv3x3(h_in, base + 0, True)
        out2 = conv3x3(out1, base + 1, True)
        out3 = conv3x3(out2, base + 2, True)
        out = conv3x3(out3, base + 3, True) + out3
        out = conv3x3(out, base + 4, True) + out2
        out = conv3x3(out, base + 5, True) + out1
        out = conv3x3(out, base + 6, False) + h_in
        return out

    h = lax.fori_loop(0, n_sub, subnet, x_ref[0].astype(jnp.float32))
    o_ref[0] = h


def _pack_weights(packed_w, packed_b, L, C):
    CP = 8 * C
    taps = packed_w.reshape(L, 3, 3, C, C)
    q = jnp.arange(8)
    mask = (q[None, :, None] == q[None, None, :] +
            jnp.arange(3)[:, None, None] - 1).astype(jnp.float32)
    wd = jnp.einsum('xqp,lyxio->lyqipo', mask, taps)
    wd = wd.reshape(L, 3, CP, CP).astype(jnp.bfloat16)
    we = jnp.zeros((L, 3, 2 * C, CP), jnp.float32)
    we = we.at[:, :, 0:C, 0:C].set(taps[:, :, 0])
    we = we.at[:, :, C:2 * C, CP - C:CP].set(taps[:, :, 2])
    we = we.reshape(L, 6 * C, CP).astype(jnp.bfloat16)
    b3 = jnp.tile(packed_b, (1, 8, 1)).reshape(L, 1, CP)
    return wd, we, b3


@jax.jit
def kernel(x, packed_w, packed_b):
    N, H, W, Cin = x.shape
    C = packed_w.shape[-1]
    L = packed_b.shape[0]
    n_sub = L // 7
    S = W // 8
    CP = 8 * C

    wd, we, b3 = _pack_weights(packed_w, packed_b, L, C)
    xp = jnp.pad(x, ((0, 0), (0, 0), (0, 0), (0, C - Cin)))
    xp = xp.reshape(N, H, S, CP)

    body = functools.partial(_drnn_body, H=H, S=S, CP=CP, n_sub=n_sub)
    out = pl.pallas_call(
        body,
        out_shape=jax.ShapeDtypeStruct((N, H, S, CP), x.dtype),
        grid_spec=pltpu.PrefetchScalarGridSpec(
            num_scalar_prefetch=0,
            grid=(N,),
            in_specs=[
                pl.BlockSpec((1, H, S, CP), lambda n: (n, 0, 0, 0)),
                pl.BlockSpec(wd.shape, lambda n: (0, 0, 0, 0)),
                pl.BlockSpec(we.shape, lambda n: (0, 0, 0)),
                pl.BlockSpec(b3.shape, lambda n: (0, 0, 0)),
            ],
            out_specs=pl.BlockSpec((1, H, S, CP), lambda n: (n, 0, 0, 0)),
            scratch_shapes=[
                pltpu.VMEM((H + 2, S, CP), jnp.bfloat16),
                pltpu.VMEM((H + 2, S, 2 * C), jnp.bfloat16),
                pltpu.VMEM((H, S, 6 * C), jnp.bfloat16),
            ],
        ),
        compiler_params=pltpu.CompilerParams(
            dimension_semantics=("parallel",)),
    )(xp, wd, we, b3)
    return out.reshape(N, H, W, C)[..., :Cin]
